# Initial kernel scaffold; baseline (speedup 1.0000x reference)
import jax
import jax.numpy as jnp
from jax.experimental import pallas as pl


def kernel(x, word_table, pos_table, type_table, ln_gamma, ln_beta):
    raise NotImplementedError("write your pallas kernel here")


# speedup vs baseline: 1.0006x; 1.0006x over previous
import functools

import jax
import jax.numpy as jnp
from jax import lax
from jax.experimental import pallas as pl
from jax.experimental.pallas import tpu as pltpu
from jax.experimental.pallas import tpu_sc as plsc

_VOCAB = 1000000
_E = 32
_L = 200
_B = 4096
_EPS = 1e-12

_ROWS = _B * _L
_NC, _NS = 2, 16
_NW = _NC * _NS
_ROWS_W = _ROWS // _NW
_STEP = 128
_STEPS_W = _ROWS_W // _STEP


def _gather_body(tbl, idx, out, idx_v, rows_v, sem):
  wid = lax.axis_index("s") * _NC + lax.axis_index("c")
  row_base = wid * _ROWS_W
  pltpu.sync_copy(idx.at[pl.ds(wid * _STEPS_W, _STEPS_W)], idx_v)

  def step(j, carry):
    pltpu.async_copy(tbl.at[idx_v.at[j]], rows_v, sem).wait()
    pltpu.sync_copy(rows_v, out.at[pl.ds(row_base + j * _STEP, _STEP)])
    return carry

  lax.fori_loop(0, _STEPS_W, step, 0)


@functools.partial(
    pl.kernel,
    mesh=plsc.VectorSubcoreMesh(core_axis_name="c", subcore_axis_name="s"),
    out_type=jax.ShapeDtypeStruct((_ROWS, _E), jnp.float32),
    scratch_types=[
        pltpu.VMEM((_STEPS_W, _STEP), jnp.int32),
        pltpu.VMEM((_STEP, _E), jnp.float32),
        pltpu.SemaphoreType.DMA,
    ],
    compiler_params=pltpu.CompilerParams(use_tc_tiling_on_sc=False),
)
def _sc_gather(tbl, idx, out, idx_v, rows_v, sem):
  _gather_body(tbl, idx, out, idx_v, rows_v, sem)


_B_BLK = 32


def _ln_body(g_ref, pt_ref, gam_ref, bet_ref, o_ref):
  e = g_ref[...] + pt_ref[...]
  m = jnp.mean(e, axis=-1, keepdims=True)
  c = e - m
  var = jnp.mean(c * c, axis=-1, keepdims=True)
  normed = c * lax.rsqrt(var + _EPS)
  o_ref[...] = normed * gam_ref[...] + bet_ref[...]


def _tc_ln(gathered, pt, gamma, beta):
  grid = (_B // _B_BLK,)
  return pl.pallas_call(
      _ln_body,
      grid=grid,
      in_specs=[
          pl.BlockSpec((_B_BLK, _L, _E), lambda i: (i, 0, 0)),
          pl.BlockSpec((1, _L, _E), lambda i: (0, 0, 0)),
          pl.BlockSpec((1, 1, _E), lambda i: (0, 0, 0)),
          pl.BlockSpec((1, 1, _E), lambda i: (0, 0, 0)),
      ],
      out_specs=pl.BlockSpec((_B_BLK, _L, _E), lambda i: (i, 0, 0)),
      out_shape=jax.ShapeDtypeStruct((_B, _L, _E), jnp.float32),
  )(gathered, pt, gamma, beta)


def kernel(x, word_table, pos_table, type_table, ln_gamma, ln_beta):
  idx = x.reshape(_ROWS // _STEP, _STEP)
  gathered = _sc_gather(word_table, idx)
  pt = (pos_table + type_table[1][None, :]).reshape(1, _L, _E)
  gam = ln_gamma.reshape(1, 1, _E)
  bet = ln_beta.reshape(1, 1, _E)
  return _tc_ln(gathered.reshape(_B, _L, _E), pt, gam, bet)
